# Initial kernel scaffold; baseline (speedup 1.0000x reference)
import functools

import jax
import jax.numpy as jnp
from jax import lax
from jax.experimental import pallas as pl
from jax.experimental.pallas import tpu as pltpu
from jax.experimental.pallas import tpu_sc as plsc

F32 = jnp.float32
NC = 2
NS = 16
NW = NC * NS
CH = 80
REG_COEF = 0.01


def _sc_mesh():
    return plsc.VectorSubcoreMesh(core_axis_name="c", subcore_axis_name="s")


@functools.lru_cache(maxsize=None)
def _deg_kernel(NP, NCH):
    @functools.partial(
        pl.kernel,
        out_type=jax.ShapeDtypeStruct((NC, NP), F32),
        mesh=_sc_mesh(),
        scratch_types=[
            pltpu.VMEM_SHARED((NP,), F32),
            pltpu.VMEM((NCH, CH), jnp.int32),
            pltpu.VMEM((CH,), F32),
        ],
    )
    def deg_kernel(dst_hbm, zn_hbm, ones_hbm, out_hbm, deg_s, idx_v, ones_v):
        c = lax.axis_index("c")
        s = lax.axis_index("s")
        w = c * NS + s
        rz = NP // NS
        pltpu.sync_copy(zn_hbm.at[pl.ds(s * rz, rz)], deg_s.at[pl.ds(s * rz, rz)])
        pltpu.sync_copy(ones_hbm, ones_v)
        pltpu.sync_copy(dst_hbm.at[pl.ds(w * NCH, NCH)], idx_v)
        plsc.subcore_barrier()

        def chunk(j, carry):
            pltpu.sync_copy(ones_v, deg_s.at[idx_v.at[j]], add=True)
            return carry

        lax.fori_loop(0, NCH, chunk, 0)
        plsc.subcore_barrier()
        pltpu.sync_copy(deg_s.at[pl.ds(s * rz, rz)], out_hbm.at[c, pl.ds(s * rz, rz)])

    return deg_kernel


@functools.lru_cache(maxsize=None)
def _agg_kernel(NP, W, NCH):
    @functools.partial(
        pl.kernel,
        out_type=jax.ShapeDtypeStruct((NC, NP, W), F32),
        mesh=_sc_mesh(),
        scratch_types=[
            pltpu.VMEM_SHARED((NP, W), F32),
            pltpu.VMEM((NCH, CH), jnp.int32),
            pltpu.VMEM((NCH, CH), jnp.int32),
            pltpu.VMEM((CH, W), F32),
        ],
    )
    def agg_kernel(src_hbm, dst_hbm, z_hbm, tab_hbm, out_hbm, agg_s, sidx, didx, buf):
        c = lax.axis_index("c")
        s = lax.axis_index("s")
        w = c * NS + s
        rz = NP // NS
        pltpu.sync_copy(z_hbm.at[pl.ds(s * rz, rz)], agg_s.at[pl.ds(s * rz, rz)])
        pltpu.sync_copy(src_hbm.at[pl.ds(w * NCH, NCH)], sidx)
        pltpu.sync_copy(dst_hbm.at[pl.ds(w * NCH, NCH)], didx)
        plsc.subcore_barrier()

        def chunk(j, carry):
            pltpu.sync_copy(tab_hbm.at[sidx.at[j]], buf)
            pltpu.sync_copy(buf, agg_s.at[didx.at[j]], add=True)
            return carry

        lax.fori_loop(0, NCH, chunk, 0)
        plsc.subcore_barrier()
        pltpu.sync_copy(agg_s.at[pl.ds(s * rz, rz)], out_hbm.at[c, pl.ds(s * rz, rz)])

    return agg_kernel


@functools.lru_cache(maxsize=None)
def _ff_kernel(NP, W, C, NCH):
    @functools.partial(
        pl.kernel,
        out_type=jax.ShapeDtypeStruct((NW, 16), F32),
        mesh=_sc_mesh(),
        scratch_types=[
            pltpu.VMEM((NCH, CH), jnp.int32),
            pltpu.VMEM((NCH, CH), jnp.int32),
            pltpu.VMEM((NCH, CH), F32),
            pltpu.VMEM((CH, W), F32),
            pltpu.VMEM((CH, W), F32),
            pltpu.VMEM((16,), F32),
        ],
    )
    def ff_kernel(src_hbm, dst_hbm, pred_hbm, fx_hbm, out_hbm,
                  sidx, didx, pbuf, buf_a, buf_b, acc_v):
        c = lax.axis_index("c")
        s = lax.axis_index("s")
        w = c * NS + s
        pltpu.sync_copy(src_hbm.at[pl.ds(w * NCH, NCH)], sidx)
        pltpu.sync_copy(dst_hbm.at[pl.ds(w * NCH, NCH)], didx)
        pltpu.sync_copy(pred_hbm.at[pl.ds(w * NCH, NCH)], pbuf)
        lanes = lax.iota(jnp.int32, 16)

        def chunk(j, acc):
            pltpu.sync_copy(fx_hbm.at[sidx.at[j]], buf_a)
            pltpu.sync_copy(fx_hbm.at[didx.at[j]], buf_b)
            jv = jnp.full((16,), j, jnp.int32)

            def grp(g, acc):
                rows = g * 16 + lanes
                ff = jnp.zeros((16,), F32)
                for col in range(C):
                    ci = jnp.full((16,), col, jnp.int32)
                    a = plsc.load_gather(buf_a, [rows, ci])
                    b = plsc.load_gather(buf_b, [rows, ci])
                    ff = ff + a * b
                p = plsc.load_gather(pbuf, [jv, rows])
                d = ff - p
                return acc + d * d

            return lax.fori_loop(0, CH // 16, grp, acc)

        acc = lax.fori_loop(0, NCH, chunk, jnp.zeros((16,), F32))
        acc_v[...] = acc
        pltpu.sync_copy(acc_v, out_hbm.at[w])

    return ff_kernel


def _mm1_call(degr, x_p, W1, NP, R):
    G = NP // R
    D = x_p.shape[1]
    H = W1.shape[1]

    def body(dga, dgb, x_ref, w_ref, dinv_ref, hs_ref):
        deg = dga[0, 0] + dgb[0, 0] + 1.0
        dinv = lax.rsqrt(deg)
        h = jnp.dot(x_ref[...], w_ref[...], preferred_element_type=F32)
        dinv_ref[0] = dinv
        hs_ref[...] = h * dinv

    return pl.pallas_call(
        body,
        grid=(G,),
        in_specs=[
            pl.BlockSpec((1, 1, R, 1), lambda i: (0, i, 0, 0)),
            pl.BlockSpec((1, 1, R, 1), lambda i: (1, i, 0, 0)),
            pl.BlockSpec((R, D), lambda i: (i, 0)),
            pl.BlockSpec((D, H), lambda i: (0, 0)),
        ],
        out_specs=[
            pl.BlockSpec((1, R, 1), lambda i: (i, 0, 0)),
            pl.BlockSpec((R, H), lambda i: (i, 0)),
        ],
        out_shape=[
            jax.ShapeDtypeStruct((G, R, 1), F32),
            jax.ShapeDtypeStruct((NP, H), F32),
        ],
    )(degr, degr, x_p, W1)


def _mm2_call(agg1, hs1, dinv, W2p, b1r, NP, R):
    G = NP // R
    H = hs1.shape[1]

    def body(ag0, ag1, hs1_ref, dinv_ref, w_ref, b_ref, hs2_ref):
        d = dinv_ref[0]
        a = ag0[0] + ag1[0] + hs1_ref[...]
        h1 = jnp.maximum(a * d + b_ref[...], 0.0)
        g = jnp.dot(h1, w_ref[...], preferred_element_type=F32)
        hs2_ref[...] = g * d

    return pl.pallas_call(
        body,
        grid=(G,),
        in_specs=[
            pl.BlockSpec((1, R, H), lambda i: (0, i, 0)),
            pl.BlockSpec((1, R, H), lambda i: (1, i, 0)),
            pl.BlockSpec((R, H), lambda i: (i, 0)),
            pl.BlockSpec((1, R, 1), lambda i: (i, 0, 0)),
            pl.BlockSpec((H, H), lambda i: (0, 0)),
            pl.BlockSpec((1, H), lambda i: (0, 0)),
        ],
        out_specs=pl.BlockSpec((R, H), lambda i: (i, 0)),
        out_shape=jax.ShapeDtypeStruct((NP, H), F32),
    )(agg1, agg1, hs1, dinv, W2p, b1r)


def _soft_call(agg2, hs2, dinv, b2p, NP, R, Nreal):
    G = NP // R
    H = hs2.shape[1]

    def body(ag0, ag1, hs2_ref, dinv_ref, b_ref, fx_ref, cs_ref):
        i = pl.program_id(0)
        d = dinv_ref[0]
        logits = (ag0[0] + ag1[0] + hs2_ref[...]) * d + b_ref[...]
        m = jnp.max(logits, axis=-1, keepdims=True)
        e = jnp.exp(logits - m)
        fx = e / jnp.sum(e, axis=-1, keepdims=True)
        rows = i * R + lax.broadcasted_iota(jnp.int32, (R, 1), 0)
        fx = jnp.where(rows < Nreal, fx, 0.0)
        fx_ref[...] = fx
        part = jnp.sum(jnp.log(1.0 - fx * fx), axis=0, keepdims=True)

        @pl.when(i == 0)
        def _():
            cs_ref[...] = part

        @pl.when(i != 0)
        def _():
            cs_ref[...] += part

    return pl.pallas_call(
        body,
        grid=(G,),
        in_specs=[
            pl.BlockSpec((1, R, H), lambda i: (0, i, 0)),
            pl.BlockSpec((1, R, H), lambda i: (1, i, 0)),
            pl.BlockSpec((R, H), lambda i: (i, 0)),
            pl.BlockSpec((1, R, 1), lambda i: (i, 0, 0)),
            pl.BlockSpec((1, H), lambda i: (0, 0)),
        ],
        out_specs=[
            pl.BlockSpec((R, H), lambda i: (i, 0)),
            pl.BlockSpec((1, H), lambda i: (0, 0)),
        ],
        out_shape=[
            jax.ShapeDtypeStruct((NP, H), F32),
            jax.ShapeDtypeStruct((1, H), F32),
        ],
    )(agg2, agg2, hs2, dinv, b2p)


def _loss_call(ffp, csum, E, C):
    H = csum.shape[1]

    def body(ff_ref, cs_ref, out_ref):
        ffsum = jnp.sum(ff_ref[...])
        cmask = lax.broadcasted_iota(jnp.int32, (1, H), 1) < C
        reg = -jnp.sum(jnp.where(cmask, jnp.log(1.0001 - jnp.exp(cs_ref[...])), 0.0))
        out_ref[0, 0] = ffsum * (1.0 / E) + REG_COEF * reg

    return pl.pallas_call(
        body,
        grid=(1,),
        in_specs=[
            pl.BlockSpec((NW, 16), lambda i: (0, 0)),
            pl.BlockSpec((1, H), lambda i: (0, 0)),
        ],
        out_specs=pl.BlockSpec(memory_space=pltpu.SMEM),
        out_shape=jax.ShapeDtypeStruct((1, 1), F32),
    )(ffp, csum)


def kernel(x, edge_index, edge_pred, W1, b1, W2, b2):
    N, D = x.shape
    H = W1.shape[1]
    C = W2.shape[1]
    E = edge_pred.shape[0]
    R = 1024
    NP = ((N + R - 1) // R) * R
    ECH = E // CH
    NCH = E // (NW * CH)

    src = edge_index[0].reshape(ECH, CH)
    dst = edge_index[1].reshape(ECH, CH)
    pred = edge_pred.reshape(ECH, CH)
    x_p = jnp.pad(x, ((0, NP - N), (0, 0)))
    zn = jnp.zeros((NP,), F32)
    z32 = jnp.zeros((NP, H), F32)
    ones_v = jnp.ones((CH,), F32)
    W2p = jnp.pad(W2, ((0, 0), (0, H - C)))
    b1r = b1.reshape(1, H)
    b2p = jnp.concatenate([b2, jnp.full((H - C,), -1e30, F32)]).reshape(1, H)

    degp = _deg_kernel(NP, NCH)(dst, zn, ones_v)
    degr = degp.reshape(NC, NP // R, R, 1)
    dinv, hs1 = _mm1_call(degr, x_p, W1, NP, R)
    agg1 = _agg_kernel(NP, H, NCH)(src, dst, z32, hs1)
    hs2 = _mm2_call(agg1, hs1, dinv, W2p, b1r, NP, R)
    agg2 = _agg_kernel(NP, H, NCH)(src, dst, z32, hs2)
    fxp, csum = _soft_call(agg2, hs2, dinv, b2p, NP, R, N)
    ffp = _ff_kernel(NP, H, C, NCH)(src, dst, pred, fxp)
    lossm = _loss_call(ffp, csum, E, C)
    return fxp[:N, :C], lossm[0, 0]


# speedup vs baseline: 30.4263x; 30.4263x over previous
import functools

import jax
import jax.numpy as jnp
from jax import lax
from jax.experimental import pallas as pl
from jax.experimental.pallas import tpu as pltpu
from jax.experimental.pallas import tpu_sc as plsc

F32 = jnp.float32
NC = 2
NS = 16
NW = NC * NS
CH = 128
REG_COEF = 0.01


def _sc_mesh():
    return plsc.VectorSubcoreMesh(core_axis_name="c", subcore_axis_name="s")


DW = 16


@functools.lru_cache(maxsize=None)
def _deg_kernel(NP, NCH):
    @functools.partial(
        pl.kernel,
        out_type=jax.ShapeDtypeStruct((NC, NP, DW), F32),
        mesh=_sc_mesh(),
        compiler_params=pltpu.CompilerParams(use_tc_tiling_on_sc=False, needs_layout_passes=False),
        scratch_types=[
            pltpu.VMEM_SHARED((NP, DW), F32),
            pltpu.VMEM((NCH, CH), jnp.int32),
            pltpu.VMEM((CH, DW), F32),
            pltpu.VMEM((NP // NS, DW), F32),
        ],
    )
    def deg_kernel(dst_hbm, out_hbm, deg_s, idx_v, ones_v, zbuf):
        c = lax.axis_index("c")
        s = lax.axis_index("s")
        w = c * NS + s
        rz = NP // NS
        lanes = lax.iota(jnp.int32, 16)
        one16 = jnp.ones((16,), F32)
        z16 = jnp.zeros((16,), F32)

        def zrow(r, carry):
            rv = jnp.full((16,), r, jnp.int32)
            plsc.store_scatter(zbuf, [rv, lanes], z16)
            return carry

        lax.fori_loop(0, rz, zrow, 0)

        def orow(r, carry):
            rv = jnp.full((16,), r, jnp.int32)
            plsc.store_scatter(ones_v, [rv, lanes], one16)
            return carry

        lax.fori_loop(0, CH, orow, 0)
        pltpu.sync_copy(zbuf, deg_s.at[pl.ds(s * rz, rz)])
        pltpu.sync_copy(dst_hbm.at[pl.ds(w * NCH, NCH)], idx_v)
        plsc.subcore_barrier()

        def chunk(j, carry):
            pltpu.sync_copy(ones_v, deg_s.at[idx_v.at[j]], add=True)
            return carry

        lax.fori_loop(0, NCH, chunk, 0)
        plsc.subcore_barrier()
        pltpu.sync_copy(deg_s.at[pl.ds(s * rz, rz)], out_hbm.at[c, pl.ds(s * rz, rz)])

    return deg_kernel


@functools.lru_cache(maxsize=None)
def _agg_kernel(NP, W, NCH):
    @functools.partial(
        pl.kernel,
        out_type=jax.ShapeDtypeStruct((NC, NP, W), F32),
        mesh=_sc_mesh(),
        compiler_params=pltpu.CompilerParams(use_tc_tiling_on_sc=False, needs_layout_passes=False),
        scratch_types=[
            pltpu.VMEM_SHARED((NP, W), F32),
            pltpu.VMEM_SHARED((NP, W), F32),
            pltpu.VMEM((NCH, CH), jnp.int32),
            pltpu.VMEM((NCH, CH), jnp.int32),
            pltpu.VMEM((CH, W), F32),
            pltpu.VMEM((CH, W), F32),
            pltpu.VMEM((NP // NS, W), F32),
            pltpu.SemaphoreType.DMA,
            pltpu.SemaphoreType.DMA,
        ],
    )
    def agg_kernel(src_hbm, dst_hbm, tab_hbm, out_hbm, agg_s, tab_s, sidx, didx,
                   buf0, buf1, zbuf, sem0, sem1):
        c = lax.axis_index("c")
        s = lax.axis_index("s")
        w = c * NS + s
        rz = NP // NS
        lanes = lax.iota(jnp.int32, 16)
        z16 = jnp.zeros((16,), F32)

        def zrow(r, carry):
            rv = jnp.full((16,), r, jnp.int32)
            for k in range(W // 16):
                plsc.store_scatter(zbuf, [rv, k * 16 + lanes], z16)
            return carry

        lax.fori_loop(0, rz, zrow, 0)
        pltpu.sync_copy(zbuf, agg_s.at[pl.ds(s * rz, rz)])
        pltpu.sync_copy(tab_hbm.at[pl.ds(s * rz, rz)], tab_s.at[pl.ds(s * rz, rz)])
        pltpu.sync_copy(src_hbm.at[pl.ds(w * NCH, NCH)], sidx)
        pltpu.sync_copy(dst_hbm.at[pl.ds(w * NCH, NCH)], didx)
        plsc.subcore_barrier()

        bufs, sems = (buf0, buf1), (sem0, sem1)
        pltpu.async_copy(tab_s.at[sidx.at[0]], buf0, sem0)
        pltpu.async_copy(tab_s.at[sidx.at[1]], buf1, sem1)

        def chunk(jj, carry):
            for b in range(2):
                j = jj * 2 + b
                pltpu.make_async_copy(tab_s.at[sidx.at[j]], bufs[b], sems[b]).wait()
                pltpu.sync_copy(bufs[b], agg_s.at[didx.at[j]], add=True)

                @pl.when(j + 2 < NCH)
                def _():
                    pltpu.async_copy(tab_s.at[sidx.at[j + 2]], bufs[b], sems[b])
            return carry

        lax.fori_loop(0, NCH // 2, chunk, 0)
        plsc.subcore_barrier()
        pltpu.sync_copy(agg_s.at[pl.ds(s * rz, rz)], out_hbm.at[c, pl.ds(s * rz, rz)])

    return agg_kernel


CG = 4
SB = 40


@functools.lru_cache(maxsize=None)
def _ff_kernel(NP, GW, ECH):
    SH = NW // CG
    RS = ECH // SH
    NB = RS // SB

    @functools.partial(
        pl.kernel,
        out_type=jax.ShapeDtypeStruct((CG, ECH, CH), F32),
        mesh=_sc_mesh(),
        compiler_params=pltpu.CompilerParams(use_tc_tiling_on_sc=False, needs_layout_passes=False),
        scratch_types=(
            [pltpu.VMEM((NP, GW), F32)]
            + [pltpu.VMEM((SB, CH), jnp.int32)] * 4
            + [pltpu.VMEM((SB, CH), F32)]
            + [pltpu.SemaphoreType.DMA] * 4
        ),
    )
    def ff_kernel(src_hbm, dst_hbm, f0, f1, f2, f3, out_hbm,
                  tab, sb0, sb1, db0, db1, ob, ss0, ss1, ds0, ds1):
        c = lax.axis_index("c")
        s = lax.axis_index("s")
        w = c * NS + s
        g = w % CG
        r0 = (w // CG) * RS
        lanes = lax.iota(jnp.int32, 16)
        sbufs, dbufs = (sb0, sb1), (db0, db1)
        ssems, dsems = (ss0, ss1), (ds0, ds1)

        for k, fk in enumerate((f0, f1, f2, f3)):
            @pl.when(g == k)
            def _(fk=fk):
                pltpu.sync_copy(fk, tab)

        pltpu.async_copy(src_hbm.at[pl.ds(r0, SB)], sb0, ss0)
        pltpu.async_copy(dst_hbm.at[pl.ds(r0, SB)], db0, ds0)

        def block(nn, carry):
            for p in range(2):
                nb = nn * 2 + p
                pltpu.make_async_copy(src_hbm.at[pl.ds(r0, SB)], sbufs[p], ssems[p]).wait()
                pltpu.make_async_copy(dst_hbm.at[pl.ds(r0, SB)], dbufs[p], dsems[p]).wait()

                @pl.when(nb + 1 < NB)
                def _(p=p, nb=nb):
                    nxt = r0 + (nb + 1) * SB
                    pltpu.async_copy(src_hbm.at[pl.ds(nxt, SB)], sbufs[1 - p], ssems[1 - p])
                    pltpu.async_copy(dst_hbm.at[pl.ds(nxt, SB)], dbufs[1 - p], dsems[1 - p])

                def row(r, carry, p=p):
                    rv = jnp.full((16,), r, jnp.int32)
                    for k in range(CH // 16):
                        cl = k * 16 + lanes
                        sv = plsc.load_gather(sbufs[p], [rv, cl])
                        dv = plsc.load_gather(dbufs[p], [rv, cl])
                        ff = jnp.zeros((16,), F32)
                        for col in range(GW):
                            cv = jnp.full((16,), col, jnp.int32)
                            a = plsc.load_gather(tab, [sv, cv])
                            bv = plsc.load_gather(tab, [dv, cv])
                            ff = ff + a * bv
                        plsc.store_scatter(ob, [rv, cl], ff)
                    return carry

                lax.fori_loop(0, SB, row, 0)
                pltpu.sync_copy(ob, out_hbm.at[g, pl.ds(r0 + nb * SB, SB)])
            return carry

        lax.fori_loop(0, NB // 2, block, 0)

    return ff_kernel


def _mm1_call(degr, x_p, W1, NP, R):
    G = NP // R
    D = x_p.shape[1]
    H = W1.shape[1]

    def body(dga, dgb, x_ref, w_ref, dinv_ref, hs_ref):
        deg = dga[0, 0][:, 0:1] + dgb[0, 0][:, 0:1] + 1.0
        dinv = lax.rsqrt(deg)
        h = jnp.dot(x_ref[...], w_ref[...], preferred_element_type=F32)
        dinv_ref[0] = dinv
        hs_ref[...] = h * dinv

    return pl.pallas_call(
        body,
        grid=(G,),
        in_specs=[
            pl.BlockSpec((1, 1, R, DW), lambda i: (0, i, 0, 0)),
            pl.BlockSpec((1, 1, R, DW), lambda i: (1, i, 0, 0)),
            pl.BlockSpec((R, D), lambda i: (i, 0)),
            pl.BlockSpec((D, H), lambda i: (0, 0)),
        ],
        out_specs=[
            pl.BlockSpec((1, R, 1), lambda i: (i, 0, 0)),
            pl.BlockSpec((R, H), lambda i: (i, 0)),
        ],
        out_shape=[
            jax.ShapeDtypeStruct((G, R, 1), F32),
            jax.ShapeDtypeStruct((NP, H), F32),
        ],
    )(degr, degr, x_p, W1)


def _mm2_call(agg1, hs1, dinv, W2p, b1r, NP, R):
    G = NP // R
    H = hs1.shape[1]

    def body(ag0, ag1, hs1_ref, dinv_ref, w_ref, b_ref, hs2_ref):
        d = dinv_ref[0]
        a = ag0[0] + ag1[0] + hs1_ref[...]
        h1 = jnp.maximum(a * d + b_ref[...], 0.0)
        g = jnp.dot(h1, w_ref[...], preferred_element_type=F32)
        hs2_ref[...] = g * d

    return pl.pallas_call(
        body,
        grid=(G,),
        in_specs=[
            pl.BlockSpec((1, R, H), lambda i: (0, i, 0)),
            pl.BlockSpec((1, R, H), lambda i: (1, i, 0)),
            pl.BlockSpec((R, H), lambda i: (i, 0)),
            pl.BlockSpec((1, R, 1), lambda i: (i, 0, 0)),
            pl.BlockSpec((H, H), lambda i: (0, 0)),
            pl.BlockSpec((1, H), lambda i: (0, 0)),
        ],
        out_specs=pl.BlockSpec((R, H), lambda i: (i, 0)),
        out_shape=jax.ShapeDtypeStruct((NP, H), F32),
    )(agg1, agg1, hs1, dinv, W2p, b1r)


def _soft_call(agg2, hs2, dinv, b2p, NP, R, Nreal):
    G = NP // R
    H = hs2.shape[1]

    GW = H // CG

    def body(ag0, ag1, hs2_ref, dinv_ref, b_ref, fx_ref, cs_ref, *f_refs):
        i = pl.program_id(0)
        d = dinv_ref[0]
        logits = (ag0[0] + ag1[0] + hs2_ref[...]) * d + b_ref[...]
        m = jnp.max(logits, axis=-1, keepdims=True)
        e = jnp.exp(logits - m)
        fx = e / jnp.sum(e, axis=-1, keepdims=True)
        rows = i * R + lax.broadcasted_iota(jnp.int32, (R, 1), 0)
        fx = jnp.where(rows < Nreal, fx, 0.0)
        fx_ref[...] = fx
        for k in range(CG):
            f_refs[k][...] = fx[:, k * GW:(k + 1) * GW]
        part = jnp.sum(jnp.log(1.0 - fx * fx), axis=0, keepdims=True)

        @pl.when(i == 0)
        def _():
            cs_ref[...] = part

        @pl.when(i != 0)
        def _():
            cs_ref[...] += part

    return pl.pallas_call(
        body,
        grid=(G,),
        in_specs=[
            pl.BlockSpec((1, R, H), lambda i: (0, i, 0)),
            pl.BlockSpec((1, R, H), lambda i: (1, i, 0)),
            pl.BlockSpec((R, H), lambda i: (i, 0)),
            pl.BlockSpec((1, R, 1), lambda i: (i, 0, 0)),
            pl.BlockSpec((1, H), lambda i: (0, 0)),
        ],
        out_specs=[
            pl.BlockSpec((R, H), lambda i: (i, 0)),
            pl.BlockSpec((1, H), lambda i: (0, 0)),
        ] + [pl.BlockSpec((R, GW), lambda i: (i, 0))] * CG,
        out_shape=[
            jax.ShapeDtypeStruct((NP, H), F32),
            jax.ShapeDtypeStruct((1, H), F32),
        ] + [jax.ShapeDtypeStruct((NP, GW), F32)] * CG,
    )(agg2, agg2, hs2, dinv, b2p)


def _loss_call(part, pred, csum, E, C):
    H = csum.shape[1]
    ECH = part.shape[1]

    def body(pa_ref, pr_ref, cs_ref, out_ref):
        pa = pa_ref[...]
        ff = pa[0] + pa[1] + pa[2] + pa[3]
        d = ff - pr_ref[...]
        ffsum = jnp.sum(d * d)
        cmask = lax.broadcasted_iota(jnp.int32, (1, H), 1) < C
        reg = -jnp.sum(jnp.where(cmask, jnp.log(1.0001 - jnp.exp(cs_ref[...])), 0.0))
        out_ref[0, 0] = ffsum * (1.0 / E) + REG_COEF * reg

    return pl.pallas_call(
        body,
        grid=(1,),
        in_specs=[
            pl.BlockSpec((CG, ECH, CH), lambda i: (0, 0, 0)),
            pl.BlockSpec((ECH, CH), lambda i: (0, 0)),
            pl.BlockSpec((1, H), lambda i: (0, 0)),
        ],
        out_specs=pl.BlockSpec(memory_space=pltpu.SMEM),
        out_shape=jax.ShapeDtypeStruct((1, 1), F32),
    )(part, pred, csum)


def kernel(x, edge_index, edge_pred, W1, b1, W2, b2):
    N, D = x.shape
    H = W1.shape[1]
    C = W2.shape[1]
    E = edge_pred.shape[0]
    R = 1024
    NP = ((N + R - 1) // R) * R
    NCH = -(-(E // (NW * CH)) // 8) * 8
    EP = NW * NCH * CH
    ECH = EP // CH

    src = jnp.concatenate([edge_index[0], jnp.full((EP - E,), N, jnp.int32)])
    dst = jnp.concatenate([edge_index[1], jnp.full((EP - E,), N, jnp.int32)])
    src = src.reshape(ECH, CH)
    dst = dst.reshape(ECH, CH)
    pred = jnp.concatenate([edge_pred, jnp.zeros((EP - E,), F32)]).reshape(ECH, CH)
    x_p = jnp.pad(x, ((0, NP - N), (0, 0)))
    W2p = jnp.pad(W2, ((0, 0), (0, H - C)))
    b1r = b1.reshape(1, H)
    b2p = jnp.concatenate([b2, jnp.full((H - C,), -1e30, F32)]).reshape(1, H)

    degp = _deg_kernel(NP, NCH)(dst)
    degr = degp.reshape(NC, NP // R, R, DW)
    dinv, hs1 = _mm1_call(degr, x_p, W1, NP, R)
    agg1 = _agg_kernel(NP, H, NCH)(src, dst, hs1)
    hs2 = _mm2_call(agg1, hs1, dinv, W2p, b1r, NP, R)
    agg2 = _agg_kernel(NP, H, NCH)(src, dst, hs2)
    fxp, csum, f0, f1, f2, f3 = _soft_call(agg2, hs2, dinv, b2p, NP, R, N)
    part = _ff_kernel(NP, H // CG, ECH)(src, dst, f0, f1, f2, f3)
    lossm = _loss_call(part, pred, csum, E, C)
    return fxp[:N, :C], lossm[0, 0]
